# baseline (speedup 1.0000x reference)
import functools

import jax
import jax.numpy as jnp
import numpy as np
from jax import lax
from jax.experimental import pallas as pl
from jax.experimental.pallas import tpu as pltpu
from jax.experimental.pallas import tpu_sc as plsc

_SR = 2
_S = 2 * _SR + 1
_N_IN = _S * _S


def _spiral_prio():
    coords = [(0, 0)]
    j = i = 0
    step = 1
    dirs = [(0, 1), (1, 0), (0, -1), (-1, 0)]
    d = 0
    while len(coords) < _S * _S:
        for _ in range(2):
            dj, di = dirs[d]
            for _ in range(step):
                j += dj
                i += di
                if abs(j) <= _SR and abs(i) <= _SR and len(coords) < _S * _S:
                    coords.append((j, i))
            d = (d + 1) % 4
        step += 1
    order = [(jj + _SR) * _S + (ii + _SR) for jj, ii in coords]
    prio = [0] * _N_IN
    for r, n in enumerate(order):
        prio[n] = r
    return prio


_PRIO = _spiral_prio()

_L = 16
_NW = 32


_NSPLIT = (13, 13)
_NPH = len(_NSPLIT)


def _body(pcode_hbm, w1_hbm, w2_hbm, vv_o, msk_o, mcv_o, tmp_o,
          *rest, n_cand, k2, w, rows_per_tec):
    pcode_v = rest[0]
    bufs = [(rest[1 + 3 * p], rest[2 + 3 * p], rest[3 + 3 * p])
            for p in range(_NPH)]
    best_v, vv_v, msk_v, mcv_v, tmp_v = rest[1 + 3 * _NPH:]
    n_lo = tuple(sum(_NSPLIT[:p]) for p in range(_NPH))
    wid = lax.axis_index("s") * 2 + lax.axis_index("c")
    iota = lax.broadcasted_iota(jnp.int32, (_L,), 0)
    pltpu.sync_copy(pcode_hbm, pcode_v)

    def start_in(row, ph):
        bh = wid * rows_per_tec + row
        b1, b2, sem = bufs[ph]
        r0 = n_lo[ph] * k2
        nr = _NSPLIT[ph] * k2
        pltpu.make_async_copy(
            w1_hbm.at[bh, pl.ds(r0, nr), :], b1, sem).start()
        pltpu.make_async_copy(
            w2_hbm.at[bh, pl.ds(r0, nr), :], b2, sem).start()

    def wait_in(ph):
        b1, b2, sem = bufs[ph]
        nr = _NSPLIT[ph] * k2
        pltpu.make_async_copy(
            w1_hbm.at[0, pl.ds(0, nr), :], b1, sem).wait()
        pltpu.make_async_copy(
            w2_hbm.at[0, pl.ds(0, nr), :], b2, sem).wait()

    code_lo = pcode_v[pl.ds(0, _L)]
    code_hi = pcode_v[pl.ds(_L, _L)]

    def code_of(n):
        bn = jnp.broadcast_to(n, (_L,))
        g_lo = code_lo.at[bn & (_L - 1)].get(mode="promise_in_bounds")
        g_hi = code_hi.at[bn & (_L - 1)].get(mode="promise_in_bounds")
        return jnp.where(bn < _L, g_lo, g_hi)

    def acc16(b1, b2, r, wl):
        sl = pl.ds(wl, _L)
        pa = [jnp.abs(b1[r + k, sl] - b2[r + k, sl]) for k in range(4)]
        for k in range(4, k2):
            pa[k & 3] = pa[k & 3] + jnp.abs(b1[r + k, sl] - b2[r + k, sl])
        return (pa[0] + pa[1]) + (pa[2] + pa[3])

    def cost_min(b1, b2, wls, n_lo, n_hi, inits):
        def nbody(n, bests):
            r = (n - n_lo) * k2
            cd = code_of(n)
            return tuple(
                jnp.minimum(best, acc16(b1, b2, r, wl) + cd)
                for wl, best in zip(wls, bests))

        return lax.fori_loop(n_lo, n_hi, nbody, tuple(inits))

    _GP = 2

    def compute_ph(ph):
        b1, b2, _ = bufs[ph]
        lo = n_lo[ph]
        hi = lo + _NSPLIT[ph] - (
            1 if ph == _NPH - 1 and n_cand > _N_IN else 0)
        for g in range(0, w // _L, _GP):
            wls = [(g + j) * _L for j in range(_GP)]
            if ph == 0:
                inits = [jnp.full((_L,), 3.0e7, jnp.float32)] * _GP
            else:
                inits = [best_v[pl.ds(wl, _L)] for wl in wls]
            bests = cost_min(b1, b2, wls, lo, hi, inits)
            for wl, best in zip(wls, bests):
                best_i = (best * 1024.0).astype(jnp.int32)
                n_bm = best_i & 31
                cols = wl + iota
                if ph < _NPH - 1:
                    best_v[pl.ds(wl, _L)] = best
                    rows = (n_bm - lo) * k2
                    if ph == 0:
                        for k in range(k2):
                            val = plsc.load_gather(b1, [rows + k, cols])
                            tmp_v[pl.ds(k * w + wl, _L)] = val
                    else:
                        in_ph = n_bm >= lo
                        rows = jnp.maximum(rows, 0)
                        for k in range(k2):
                            val = plsc.load_gather(
                                b1, [rows + k, cols], mask=in_ph)
                            plsc.store_scatter(
                                tmp_v, [k * w + cols], val, mask=in_ph)
                    continue
                mcb = best_i >> 10
                if n_cand > _N_IN:
                    rr = (_N_IN - lo) * k2
                    c25 = acc16(b1, b2, rr, wl).astype(jnp.int32)
                    hit = c25 < mcb
                    mskv = hit.astype(jnp.int32)
                    mcv = jnp.minimum(c25, mcb)
                    idx_tm = jnp.where(hit, _N_IN, n_bm)
                else:
                    mskv = jnp.zeros((_L,), jnp.int32)
                    mcv = mcb
                    idx_tm = n_bm
                row_i = (n_bm * 13) >> 6
                col_i = n_bm - row_i * 5
                sl = pl.ds(wl, _L)
                vv_v[sl] = (2 - row_i).astype(jnp.float32)
                vv_v[pl.ds(w + wl, _L)] = (2 - col_i).astype(jnp.float32)
                msk_v[sl] = mskv
                mcv_v[sl] = mcv
                in_ph = idx_tm >= lo
                rows = jnp.maximum((idx_tm - lo), 0) * k2
                for k in range(k2):
                    val = plsc.load_gather(b1, [rows + k, cols], mask=in_ph)
                    plsc.store_scatter(
                        tmp_v, [k * w + cols], val, mask=in_ph)

    for ph in range(_NPH - 1):
        start_in(0, ph)

    def row_loop(i, _):
        start_in(i, _NPH - 1)
        for ph in range(_NPH):
            wait_in(ph)
            compute_ph(ph)
            if ph < _NPH - 1:
                @pl.when(i + 1 < rows_per_tec)
                def _(ph=ph):
                    start_in(i + 1, ph)

        bh = wid * rows_per_tec + i
        pltpu.sync_copy(vv_v, vv_o.at[pl.ds(bh * 2 * w, 2 * w)])
        pltpu.sync_copy(msk_v, msk_o.at[pl.ds(bh * w, w)])
        pltpu.sync_copy(mcv_v, mcv_o.at[pl.ds(bh * w, w)])
        pltpu.sync_copy(tmp_v, tmp_o.at[pl.ds(bh * k2 * w, k2 * w)])
        return ()

    lax.fori_loop(0, rows_per_tec, row_loop, ())


_RSC = 288


def _tc_body(w1_ref, w2_ref, vv_ref, msk_ref, mcv_ref, tmp_ref,
             *, n_cand, k2, w):
    R = w1_ref.shape[0]
    x = w1_ref[...].reshape(R, n_cand, k2, w)
    y = w2_ref[...].reshape(R, n_cand, k2, w)
    cost = jnp.sum(jnp.abs(x - y), axis=2)
    best = cost[:, 0, :] * 1024.0 + float(_PRIO[0] << 5)
    for n in range(1, _N_IN):
        key_n = cost[:, n, :] * 1024.0 + float((_PRIO[n] << 5) | n)
        best = jnp.minimum(best, key_n)
    best_i = best.astype(jnp.int32)
    n_bm = best_i & 31
    mcb = best_i >> 10
    if n_cand > _N_IN:
        c25 = cost[:, _N_IN, :].astype(jnp.int32)
        hit = c25 < mcb
        mskv = hit.astype(jnp.int32)
        mcv = jnp.minimum(c25, mcb)
        idx_tm = jnp.where(hit, _N_IN, n_bm)
    else:
        mskv = jnp.zeros_like(n_bm)
        mcv = mcb
        idx_tm = n_bm
    row_i = (n_bm * 13) >> 6
    col_i = n_bm - row_i * 5
    vy = (2 - row_i).astype(jnp.float32)
    vx = (2 - col_i).astype(jnp.float32)
    vv_ref[...] = jnp.stack([vy, vx], axis=1)
    msk_ref[...] = mskv
    mcv_ref[...] = mcv
    nids = lax.broadcasted_iota(jnp.int32, (R, n_cand, k2, w), 1)
    sel = (nids == idx_tm[:, None, None, :]).astype(jnp.float32)
    tmp_ref[...] = jnp.sum(x * sel, axis=1)


@jax.jit
def kernel(w1, w2):
    B, H, W, N, K2 = w1.shape
    BH = B * H
    rows_per_tec = _RSC // _NW
    mesh = plsc.VectorSubcoreMesh(
        core_axis_name="c", subcore_axis_name="s", num_cores=2, num_subcores=16
    )
    f32 = jnp.float32
    i32 = jnp.int32
    out_type = (
        jax.ShapeDtypeStruct((_RSC * 2 * W,), f32),
        jax.ShapeDtypeStruct((_RSC * W,), i32),
        jax.ShapeDtypeStruct((_RSC * W,), i32),
        jax.ShapeDtypeStruct((_RSC * K2 * W,), f32),
    )
    scratch = [pltpu.VMEM((_L * 2,), f32)]
    for p in range(_NPH):
        scratch.append(pltpu.VMEM((_NSPLIT[p] * K2, W), f32))
        scratch.append(pltpu.VMEM((_NSPLIT[p] * K2, W), f32))
        scratch.append(pltpu.SemaphoreType.DMA)
    scratch += [
        pltpu.VMEM((W,), f32),
        pltpu.VMEM((2 * W,), f32),
        pltpu.VMEM((W,), i32),
        pltpu.VMEM((W,), i32),
        pltpu.VMEM((K2 * W,), f32),
    ]
    scratch = tuple(scratch)
    run = pl.kernel(
        functools.partial(_body, n_cand=N, k2=K2, w=W,
                          rows_per_tec=rows_per_tec),
        out_type=out_type,
        mesh=mesh,
        scratch_types=scratch,
        compiler_params=pltpu.CompilerParams(needs_layout_passes=False),
    )
    pcode = np.full((_L * 2,), (1 << 22), np.float32)
    for n in range(_N_IN):
        pcode[n] = ((_PRIO[n] << 5) | n) / 1024.0
    w1t = jnp.transpose(w1, (0, 1, 3, 4, 2)).reshape(BH, N * K2, W)
    w2t = jnp.transpose(w2, (0, 1, 3, 4, 2)).reshape(BH, N * K2, W)
    vv, msk, mcv, tmp = run(jnp.asarray(pcode), w1t, w2t)

    NT = BH - _RSC
    RT = 8
    tc = pl.pallas_call(
        functools.partial(_tc_body, n_cand=N, k2=K2, w=W),
        grid=(NT // RT,),
        in_specs=[
            pl.BlockSpec((RT, N * K2, W), lambda i: (_RSC // RT + i, 0, 0)),
            pl.BlockSpec((RT, N * K2, W), lambda i: (_RSC // RT + i, 0, 0)),
        ],
        out_specs=[
            pl.BlockSpec((RT, 2, W), lambda i: (i, 0, 0)),
            pl.BlockSpec((RT, W), lambda i: (i, 0)),
            pl.BlockSpec((RT, W), lambda i: (i, 0)),
            pl.BlockSpec((RT, K2, W), lambda i: (i, 0, 0)),
        ],
        out_shape=[
            jax.ShapeDtypeStruct((NT, 2, W), f32),
            jax.ShapeDtypeStruct((NT, W), i32),
            jax.ShapeDtypeStruct((NT, W), i32),
            jax.ShapeDtypeStruct((NT, K2, W), f32),
        ],
    )
    vv_t, msk_t, mcv_t, tmp_t = tc(w1t, w2t)

    vv_all = jnp.concatenate([vv.reshape(_RSC, 2, W), vv_t], axis=0)
    msk_all = jnp.concatenate([msk.reshape(_RSC, W), msk_t], axis=0)
    mcv_all = jnp.concatenate([mcv.reshape(_RSC, W), mcv_t], axis=0)
    tmp_all = jnp.concatenate([tmp.reshape(_RSC, K2, W), tmp_t], axis=0)

    vector = vv_all.reshape(B, H, 2, W).transpose(0, 1, 3, 2)
    vector = vector.astype(jnp.float16)
    min_templates = tmp_all.reshape(B, H, 1, K2, W).transpose(0, 1, 4, 2, 3)
    input_mv_mask = (msk_all > 0).reshape(B, H, W, 1)
    min_cost_volume = mcv_all.reshape(B, H, W, 1)
    return (vector, min_templates, input_mv_mask, min_cost_volume)


# speedup vs baseline: 1.0973x; 1.0973x over previous
import functools

import jax
import jax.numpy as jnp
import numpy as np
from jax import lax
from jax.experimental import pallas as pl
from jax.experimental.pallas import tpu as pltpu
from jax.experimental.pallas import tpu_sc as plsc

_SR = 2
_S = 2 * _SR + 1
_N_IN = _S * _S


def _spiral_prio():
    coords = [(0, 0)]
    j = i = 0
    step = 1
    dirs = [(0, 1), (1, 0), (0, -1), (-1, 0)]
    d = 0
    while len(coords) < _S * _S:
        for _ in range(2):
            dj, di = dirs[d]
            for _ in range(step):
                j += dj
                i += di
                if abs(j) <= _SR and abs(i) <= _SR and len(coords) < _S * _S:
                    coords.append((j, i))
            d = (d + 1) % 4
        step += 1
    order = [(jj + _SR) * _S + (ii + _SR) for jj, ii in coords]
    prio = [0] * _N_IN
    for r, n in enumerate(order):
        prio[n] = r
    return prio


_PRIO = _spiral_prio()

_L = 16
_NW = 32


_NSPLIT = (13, 13)
_NPH = len(_NSPLIT)


def _body(pcode_hbm, w1_hbm, w2_hbm, vv_o, msk_o, mcv_o, tmp_o,
          *rest, n_cand, k2, w, rows_per_tec):
    pcode_v = rest[0]
    bufs = [(rest[1 + 3 * p], rest[2 + 3 * p], rest[3 + 3 * p])
            for p in range(_NPH)]
    best_v, vv_v, msk_v, mcv_v, tmp_v = rest[1 + 3 * _NPH:]
    n_lo = tuple(sum(_NSPLIT[:p]) for p in range(_NPH))
    wid = lax.axis_index("s") * 2 + lax.axis_index("c")
    iota = lax.broadcasted_iota(jnp.int32, (_L,), 0)
    pltpu.sync_copy(pcode_hbm, pcode_v)

    def start_in(row, ph):
        bh = wid * rows_per_tec + row
        b1, b2, sem = bufs[ph]
        r0 = n_lo[ph] * k2
        nr = _NSPLIT[ph] * k2
        pltpu.make_async_copy(
            w1_hbm.at[bh, pl.ds(r0, nr), :], b1, sem).start()
        pltpu.make_async_copy(
            w2_hbm.at[bh, pl.ds(r0, nr), :], b2, sem).start()

    def wait_in(ph):
        b1, b2, sem = bufs[ph]
        nr = _NSPLIT[ph] * k2
        pltpu.make_async_copy(
            w1_hbm.at[0, pl.ds(0, nr), :], b1, sem).wait()
        pltpu.make_async_copy(
            w2_hbm.at[0, pl.ds(0, nr), :], b2, sem).wait()

    code_lo = pcode_v[pl.ds(0, _L)]
    code_hi = pcode_v[pl.ds(_L, _L)]

    def code_of(n):
        bn = jnp.broadcast_to(n, (_L,))
        g_lo = code_lo.at[bn & (_L - 1)].get(mode="promise_in_bounds")
        g_hi = code_hi.at[bn & (_L - 1)].get(mode="promise_in_bounds")
        return jnp.where(bn < _L, g_lo, g_hi)

    def acc16(b1, b2, r, wl):
        sl = pl.ds(wl, _L)
        pa = [jnp.abs(b1[r + k, sl] - b2[r + k, sl]) for k in range(4)]
        for k in range(4, k2):
            pa[k & 3] = pa[k & 3] + jnp.abs(b1[r + k, sl] - b2[r + k, sl])
        return (pa[0] + pa[1]) + (pa[2] + pa[3])

    def cost_min(b1, b2, wls, n_lo, n_hi, inits):
        def nbody(n, bests):
            r = (n - n_lo) * k2
            cd = code_of(n)
            return tuple(
                jnp.minimum(best, acc16(b1, b2, r, wl) + cd)
                for wl, best in zip(wls, bests))

        return lax.fori_loop(n_lo, n_hi, nbody, tuple(inits))

    _GP = 2

    def compute_ph(ph):
        b1, b2, _ = bufs[ph]
        lo = n_lo[ph]
        hi = lo + _NSPLIT[ph] - (
            1 if ph == _NPH - 1 and n_cand > _N_IN else 0)
        for g in range(0, w // _L, _GP):
            wls = [(g + j) * _L for j in range(_GP)]
            if ph == 0:
                inits = [jnp.full((_L,), 3.0e7, jnp.float32)] * _GP
            else:
                inits = [best_v[pl.ds(wl, _L)] for wl in wls]
            bests = cost_min(b1, b2, wls, lo, hi, inits)
            for wl, best in zip(wls, bests):
                best_i = (best * 1024.0).astype(jnp.int32)
                n_bm = best_i & 31
                cols = wl + iota
                if ph < _NPH - 1:
                    best_v[pl.ds(wl, _L)] = best
                    rows = (n_bm - lo) * k2
                    if ph == 0:
                        for k in range(k2):
                            val = plsc.load_gather(b1, [rows + k, cols])
                            tmp_v[pl.ds(k * w + wl, _L)] = val
                    else:
                        in_ph = n_bm >= lo
                        rows = jnp.maximum(rows, 0)
                        for k in range(k2):
                            val = plsc.load_gather(
                                b1, [rows + k, cols], mask=in_ph)
                            plsc.store_scatter(
                                tmp_v, [k * w + cols], val, mask=in_ph)
                    continue
                mcb = best_i >> 10
                if n_cand > _N_IN:
                    rr = (_N_IN - lo) * k2
                    c25 = acc16(b1, b2, rr, wl).astype(jnp.int32)
                    hit = c25 < mcb
                    mskv = hit.astype(jnp.int32)
                    mcv = jnp.minimum(c25, mcb)
                    idx_tm = jnp.where(hit, _N_IN, n_bm)
                else:
                    mskv = jnp.zeros((_L,), jnp.int32)
                    mcv = mcb
                    idx_tm = n_bm
                row_i = (n_bm * 13) >> 6
                col_i = n_bm - row_i * 5
                sl = pl.ds(wl, _L)
                vv_v[sl] = (2 - row_i).astype(jnp.float32)
                vv_v[pl.ds(w + wl, _L)] = (2 - col_i).astype(jnp.float32)
                msk_v[sl] = mskv
                mcv_v[sl] = mcv
                in_ph = idx_tm >= lo
                rows = jnp.maximum((idx_tm - lo), 0) * k2
                for k in range(k2):
                    val = plsc.load_gather(b1, [rows + k, cols], mask=in_ph)
                    plsc.store_scatter(
                        tmp_v, [k * w + cols], val, mask=in_ph)

    for ph in range(_NPH - 1):
        start_in(0, ph)

    def row_loop(i, _):
        start_in(i, _NPH - 1)
        for ph in range(_NPH):
            wait_in(ph)
            compute_ph(ph)
            if ph < _NPH - 1:
                @pl.when(i + 1 < rows_per_tec)
                def _(ph=ph):
                    start_in(i + 1, ph)

        bh = wid * rows_per_tec + i
        pltpu.sync_copy(vv_v, vv_o.at[pl.ds(bh * 2 * w, 2 * w)])
        pltpu.sync_copy(msk_v, msk_o.at[pl.ds(bh * w, w)])
        pltpu.sync_copy(mcv_v, mcv_o.at[pl.ds(bh * w, w)])
        pltpu.sync_copy(tmp_v, tmp_o.at[pl.ds(bh * k2 * w, k2 * w)])
        return ()

    lax.fori_loop(0, rows_per_tec, row_loop, ())


_RSC = 224


def _tc_body(w1_ref, w2_ref, vv_ref, msk_ref, mcv_ref, tmp_ref,
             *, n_cand, k2, w):
    R = w1_ref.shape[0]
    x = w1_ref[...].reshape(R, n_cand, k2, w)
    y = w2_ref[...].reshape(R, n_cand, k2, w)
    cost = jnp.sum(jnp.abs(x - y), axis=2)
    best = cost[:, 0, :] * 1024.0 + float(_PRIO[0] << 5)
    for n in range(1, _N_IN):
        key_n = cost[:, n, :] * 1024.0 + float((_PRIO[n] << 5) | n)
        best = jnp.minimum(best, key_n)
    best_i = best.astype(jnp.int32)
    n_bm = best_i & 31
    mcb = best_i >> 10
    if n_cand > _N_IN:
        c25 = cost[:, _N_IN, :].astype(jnp.int32)
        hit = c25 < mcb
        mskv = hit.astype(jnp.int32)
        mcv = jnp.minimum(c25, mcb)
        idx_tm = jnp.where(hit, _N_IN, n_bm)
    else:
        mskv = jnp.zeros_like(n_bm)
        mcv = mcb
        idx_tm = n_bm
    row_i = (n_bm * 13) >> 6
    col_i = n_bm - row_i * 5
    vy = (2 - row_i).astype(jnp.float32)
    vx = (2 - col_i).astype(jnp.float32)
    vv_ref[...] = jnp.stack([vy, vx], axis=1)
    msk_ref[...] = mskv
    mcv_ref[...] = mcv
    nids = lax.broadcasted_iota(jnp.int32, (R, n_cand, k2, w), 1)
    sel = (nids == idx_tm[:, None, None, :]).astype(jnp.float32)
    tmp_ref[...] = jnp.sum(x * sel, axis=1)


@jax.jit
def kernel(w1, w2):
    B, H, W, N, K2 = w1.shape
    BH = B * H
    rows_per_tec = _RSC // _NW
    mesh = plsc.VectorSubcoreMesh(
        core_axis_name="c", subcore_axis_name="s", num_cores=2, num_subcores=16
    )
    f32 = jnp.float32
    i32 = jnp.int32
    out_type = (
        jax.ShapeDtypeStruct((_RSC * 2 * W,), f32),
        jax.ShapeDtypeStruct((_RSC * W,), i32),
        jax.ShapeDtypeStruct((_RSC * W,), i32),
        jax.ShapeDtypeStruct((_RSC * K2 * W,), f32),
    )
    scratch = [pltpu.VMEM((_L * 2,), f32)]
    for p in range(_NPH):
        scratch.append(pltpu.VMEM((_NSPLIT[p] * K2, W), f32))
        scratch.append(pltpu.VMEM((_NSPLIT[p] * K2, W), f32))
        scratch.append(pltpu.SemaphoreType.DMA)
    scratch += [
        pltpu.VMEM((W,), f32),
        pltpu.VMEM((2 * W,), f32),
        pltpu.VMEM((W,), i32),
        pltpu.VMEM((W,), i32),
        pltpu.VMEM((K2 * W,), f32),
    ]
    scratch = tuple(scratch)
    run = pl.kernel(
        functools.partial(_body, n_cand=N, k2=K2, w=W,
                          rows_per_tec=rows_per_tec),
        out_type=out_type,
        mesh=mesh,
        scratch_types=scratch,
        compiler_params=pltpu.CompilerParams(needs_layout_passes=False),
    )
    pcode = np.full((_L * 2,), (1 << 22), np.float32)
    for n in range(_N_IN):
        pcode[n] = ((_PRIO[n] << 5) | n) / 1024.0
    w1t = jnp.transpose(w1, (0, 1, 3, 4, 2)).reshape(BH, N * K2, W)
    w2t = jnp.transpose(w2, (0, 1, 3, 4, 2)).reshape(BH, N * K2, W)
    vv, msk, mcv, tmp = run(jnp.asarray(pcode), w1t, w2t)

    NT = BH - _RSC
    RT = 8
    tc = pl.pallas_call(
        functools.partial(_tc_body, n_cand=N, k2=K2, w=W),
        grid=(NT // RT,),
        in_specs=[
            pl.BlockSpec((RT, N * K2, W), lambda i: (_RSC // RT + i, 0, 0)),
            pl.BlockSpec((RT, N * K2, W), lambda i: (_RSC // RT + i, 0, 0)),
        ],
        out_specs=[
            pl.BlockSpec((RT, 2, W), lambda i: (i, 0, 0)),
            pl.BlockSpec((RT, W), lambda i: (i, 0)),
            pl.BlockSpec((RT, W), lambda i: (i, 0)),
            pl.BlockSpec((RT, K2, W), lambda i: (i, 0, 0)),
        ],
        out_shape=[
            jax.ShapeDtypeStruct((NT, 2, W), f32),
            jax.ShapeDtypeStruct((NT, W), i32),
            jax.ShapeDtypeStruct((NT, W), i32),
            jax.ShapeDtypeStruct((NT, K2, W), f32),
        ],
    )
    vv_t, msk_t, mcv_t, tmp_t = tc(w1t, w2t)

    vv_all = jnp.concatenate([vv.reshape(_RSC, 2, W), vv_t], axis=0)
    msk_all = jnp.concatenate([msk.reshape(_RSC, W), msk_t], axis=0)
    mcv_all = jnp.concatenate([mcv.reshape(_RSC, W), mcv_t], axis=0)
    tmp_all = jnp.concatenate([tmp.reshape(_RSC, K2, W), tmp_t], axis=0)

    vector = vv_all.reshape(B, H, 2, W).transpose(0, 1, 3, 2)
    vector = vector.astype(jnp.float16)
    min_templates = tmp_all.reshape(B, H, 1, K2, W).transpose(0, 1, 4, 2, 3)
    input_mv_mask = (msk_all > 0).reshape(B, H, W, 1)
    min_cost_volume = mcv_all.reshape(B, H, W, 1)
    return (vector, min_templates, input_mv_mask, min_cost_volume)
